# baseline (speedup 1.0000x reference)
import functools

import jax
import jax.numpy as jnp
from jax import lax
from jax.experimental import pallas as pl
from jax.experimental.pallas import tpu as pltpu
from jax.experimental.pallas import tpu_sc as plsc

N = 100000
E = 1600000
HID = 32

NC = 2
NS = 16
NW = NC * NS
CH = 896
EPW = 50176
E_PAD = NW * EPW
NCH_G = EPW // CH
CHS = 256
EH = E_PAD // 2
EPW_H = EPW // 2
NCH_GH = EPW_H // CH
PT_H = EH // NS
NCH_SH = PT_H // CHS
HALF = N // NC
ZROWS = 3128
ACC_ROWS = NS * ZROWS
CPR = N // NC // NS

E4 = E_PAD // 4
E4H = E4 // 2
N4 = N // 4
BE4 = 2048
BN4 = 5000

_mesh = plsc.VectorSubcoreMesh(core_axis_name="c", subcore_axis_name="s")


@functools.partial(
    pl.kernel,
    mesh=_mesh,
    compiler_params=pltpu.CompilerParams(use_tc_tiling_on_sc=False),
    out_type=(
        jax.ShapeDtypeStruct((EH, HID), jnp.float32),
        jax.ShapeDtypeStruct((EH, HID), jnp.float32),
    ),
    scratch_types=[
        pltpu.VMEM((CH,), jnp.int32),
        pltpu.VMEM((CH,), jnp.int32),
        pltpu.VMEM((CH,), jnp.int32),
        pltpu.VMEM((CH,), jnp.int32),
        pltpu.VMEM((CH, HID), jnp.float32),
        pltpu.VMEM((CH, HID), jnp.float32),
        pltpu.VMEM((CH, HID), jnp.float32),
        pltpu.VMEM((CH, HID), jnp.float32),
        pltpu.SemaphoreType.DMA,
        pltpu.SemaphoreType.DMA,
        pltpu.SemaphoreType.DMA,
        pltpu.SemaphoreType.DMA,
        pltpu.SemaphoreType.DMA,
        pltpu.SemaphoreType.DMA,
    ],
)
def _sc_gather(h_hbm, dst1_hbm, src1_hbm, ga_hbm, gb_hbm,
               idxd0, idxs0, idxd1, idxs1, ga0, gb0, ga1, gb1,
               semi0, semi1, semg0, semg1, sems0, sems1):
    c = lax.axis_index("c")
    s = lax.axis_index("s")
    wid = s * NC + c
    base = wid * EPW_H
    idxd = [idxd0, idxd1]
    idxs = [idxs0, idxs1]
    ga_v = [ga0, ga1]
    gb_v = [gb0, gb1]
    semi = [semi0, semi1]
    semg = [semg0, semg1]
    sems = [sems0, sems1]

    def fire_idx(i, b):
        e0 = base + i * CH
        pltpu.async_copy(dst1_hbm.at[pl.ds(e0, CH)], idxd[b], semi[b])
        pltpu.async_copy(src1_hbm.at[pl.ds(e0, CH)], idxs[b], semi[b])

    fire_idx(0, 0)

    def chunk(i, b):
        @pl.when(i >= 2)
        def _():
            pltpu.make_async_copy(ga_v[b], ga_hbm.at[pl.ds(0, CH)],
                                  sems[b]).wait()
            pltpu.make_async_copy(gb_v[b], gb_hbm.at[pl.ds(0, CH)],
                                  sems[b]).wait()

        @pl.when(i + 1 < NCH_GH)
        def _():
            fire_idx(i + 1, 1 - b)

        pltpu.make_async_copy(dst1_hbm.at[pl.ds(0, CH)], idxd[b],
                              semi[b]).wait()
        pltpu.make_async_copy(src1_hbm.at[pl.ds(0, CH)], idxs[b],
                              semi[b]).wait()
        for j in range(CH // 128):
            pltpu.async_copy(h_hbm.at[idxd[b].at[pl.ds(j * 128, 128)]],
                             ga_v[b].at[pl.ds(j * 128, 128)], semg[b])
        for j in range(CH // 128):
            pltpu.async_copy(h_hbm.at[idxs[b].at[pl.ds(j * 128, 128)]],
                             gb_v[b].at[pl.ds(j * 128, 128)], semg[b])
        pltpu.make_async_copy(h_hbm.at[pl.ds(0, CH)], ga_v[b],
                              semg[b]).wait()
        pltpu.make_async_copy(h_hbm.at[pl.ds(0, CH)], gb_v[b],
                              semg[b]).wait()
        e0 = base + i * CH
        pltpu.async_copy(ga_v[b], ga_hbm.at[pl.ds(e0, CH)], sems[b])
        pltpu.async_copy(gb_v[b], gb_hbm.at[pl.ds(e0, CH)], sems[b])

    def body(ii, carry):
        for b in range(2):
            chunk(ii * 2 + b, b)
        return carry

    lax.fori_loop(0, NCH_GH // 2, body, 0)
    for b in range(2):
        pltpu.make_async_copy(ga_v[b], ga_hbm.at[pl.ds(0, CH)],
                              sems[b]).wait()
        pltpu.make_async_copy(gb_v[b], gb_hbm.at[pl.ds(0, CH)],
                              sems[b]).wait()


@functools.partial(
    pl.kernel,
    mesh=_mesh,
    compiler_params=pltpu.CompilerParams(use_tc_tiling_on_sc=False),
    out_type=jax.ShapeDtypeStruct((N, HID), jnp.float32),
    scratch_types=[
        pltpu.VMEM((CHS,), jnp.int32),
        pltpu.VMEM((CHS,), jnp.int32),
        pltpu.VMEM((CHS // 128, 128), jnp.int32),
        pltpu.VMEM((CHS // 128, 128), jnp.int32),
        pltpu.VMEM((CHS, HID), jnp.float32),
        pltpu.VMEM((CHS, HID), jnp.float32),
        pltpu.VMEM_SHARED((ACC_ROWS, HID), jnp.float32),
        pltpu.SemaphoreType.DMA,
        pltpu.SemaphoreType.DMA,
        pltpu.SemaphoreType.DMA,
        pltpu.SemaphoreType.DMA,
    ],
)
def _sc_scatter(upd_hbm, src1_hbm, agg_hbm,
                idxs0, idxs1, idxl0, idxl1, rows0, rows1, acc,
                semi0, semi1, semr0, semr1):
    c = lax.axis_index("c")
    s = lax.axis_index("s")
    nbase = c * HALF
    idxs = [idxs0, idxs1]
    idxl = [idxl0, idxl1]
    rows = [rows0, rows1]
    semi = [semi0, semi1]
    semr = [semr0, semr1]

    def zbody(r, carry):
        rows0[r, pl.ds(0, 16)] = jnp.zeros((16,), jnp.float32)
        rows0[r, pl.ds(16, 16)] = jnp.zeros((16,), jnp.float32)
        return carry

    lax.fori_loop(0, CHS, zbody, 0)
    for k in range(ZROWS // CHS):
        pltpu.sync_copy(rows0, acc.at[pl.ds(s * ZROWS + k * CHS, CHS)])
    tail = ZROWS - (ZROWS // CHS) * CHS
    pltpu.sync_copy(rows0.at[pl.ds(0, tail)],
                    acc.at[pl.ds(s * ZROWS + ZROWS - tail, tail)])
    plsc.subcore_barrier()

    t0 = s * PT_H

    def fire(i, b):
        e0 = t0 + i * CHS
        pltpu.async_copy(src1_hbm.at[pl.ds(e0, CHS)], idxs[b], semi[b])
        pltpu.async_copy(upd_hbm.at[pl.ds(e0, CHS)], rows[b], semr[b])

    fire(0, 0)

    def chunk(i, b):
        @pl.when(i + 1 < NCH_SH)
        def _():
            fire(i + 1, 1 - b)

        pltpu.make_async_copy(src1_hbm.at[pl.ds(0, CHS)], idxs[b],
                              semi[b]).wait()
        for k in range(CHS // 16):
            v = idxs[b][pl.ds(k * 16, 16)]
            loc = v - nbase
            ok = (loc >= 0) & (loc < HALF)
            idxl[b][k // 8, pl.ds((k % 8) * 16, 16)] = jnp.where(ok, loc,
                                                                 HALF)
        pltpu.make_async_copy(upd_hbm.at[pl.ds(0, CHS)], rows[b],
                              semr[b]).wait()
        for j in range(CHS // 128):
            pltpu.sync_copy(rows[b].at[pl.ds(j * 128, 128)],
                            acc.at[idxl[b].at[j]], add=True)

    def body(ii, carry):
        for b in range(2):
            chunk(ii * 2 + b, b)
        return carry

    lax.fori_loop(0, NCH_SH // 2, body, 0)
    plsc.subcore_barrier()
    pltpu.sync_copy(acc.at[pl.ds(s * CPR, CPR)],
                    agg_hbm.at[pl.ds(nbase + s * CPR, CPR)])


def _ln4(u, g, beta, bdo):
    hp = jax.lax.Precision.HIGHEST
    mu = jnp.dot(u, bdo, preferred_element_type=jnp.float32, precision=hp)
    d = u - mu
    var = jnp.dot(d * d, bdo, preferred_element_type=jnp.float32,
                  precision=hp)
    return d * lax.rsqrt(var + 1e-5) * g + beta


def _full(a):
    return pl.BlockSpec(a.shape, lambda i: tuple(0 for _ in a.shape))


def _rows(block, width):
    return pl.BlockSpec((block, width), lambda i: (i, 0))


def _enc_body(x_ref, mean_ref, std_ref, W1, b1, W2, b2, g, beta, bdo,
              out_ref):
    xn = (x_ref[...] - mean_ref[...]) / std_ref[...]
    t = jnp.maximum(jnp.dot(xn, W1[...], preferred_element_type=jnp.float32)
                    + b1[...], 0.0)
    u = jnp.dot(t, W2[...], preferred_element_type=jnp.float32) + b2[...]
    out_ref[...] = _ln4(u, g[...], beta[...], bdo[...])


def _tc_encoder(arr, mean, std, w, bdo, block):
    n = arr.shape[0]
    args = (arr, mean, std, w["W1"], w["b1"], w["W2"], w["b2"], w["g"],
            w["beta"], bdo)
    return pl.pallas_call(
        _enc_body,
        grid=(n // block,),
        in_specs=[_rows(block, arr.shape[1])] + [_full(a) for a in args[1:]],
        out_specs=_rows(block, 128),
        out_shape=jax.ShapeDtypeStruct((n, 128), jnp.float32),
    )(*args)


def _edge_body(ga, gb, e, W1a, W1b, W1c, b1, W2, b2, g, beta, bdo, out_ref):
    t = (jnp.dot(ga[...], W1a[...], preferred_element_type=jnp.float32)
         + jnp.dot(gb[...], W1b[...], preferred_element_type=jnp.float32)
         + jnp.dot(e[...], W1c[...], preferred_element_type=jnp.float32)
         + b1[...])
    t = jnp.maximum(t, 0.0)
    u = jnp.dot(t, W2[...], preferred_element_type=jnp.float32) + b2[...]
    out_ref[...] = _ln4(u, g[...], beta[...], bdo[...]) + e[...]


def _tc_edge(ga, gb, e, w, bdo):
    args = (ga, gb, e, w["W1a"], w["W1b"], w["W1c"], w["b1"], w["W2"],
            w["b2"], w["g"], w["beta"], bdo)
    return pl.pallas_call(
        _edge_body,
        grid=(E4H // BE4,),
        in_specs=[_rows(BE4, 128)] * 3 + [_full(a) for a in args[3:]],
        out_specs=_rows(BE4, 128),
        out_shape=jax.ShapeDtypeStruct((E4H, 128), jnp.float32),
    )(*args)


def _node_body(h, agg1, agg2, W1a, W1b, b1, W2, b2, g, beta, bdo, out_ref):
    agg = agg1[...] + agg2[...]
    t = (jnp.dot(h[...], W1a[...], preferred_element_type=jnp.float32)
         + jnp.dot(agg, W1b[...], preferred_element_type=jnp.float32)
         + b1[...])
    t = jnp.maximum(t, 0.0)
    u = jnp.dot(t, W2[...], preferred_element_type=jnp.float32) + b2[...]
    out_ref[...] = h[...] + _ln4(u, g[...], beta[...], bdo[...])


def _tc_node(h, agg1, agg2, w, bdo):
    args = (h, agg1, agg2, w["W1a"], w["W1b"], w["b1"], w["W2"], w["b2"],
            w["g"], w["beta"], bdo)
    return pl.pallas_call(
        _node_body,
        grid=(N4 // BN4,),
        in_specs=[_rows(BN4, 128)] * 3 + [_full(a) for a in args[3:]],
        out_specs=_rows(BN4, 128),
        out_shape=jax.ShapeDtypeStruct((N4, 128), jnp.float32),
    )(*args)


def _dec_body(h, W1, b1, W2, b2, out_ref):
    t = jnp.maximum(jnp.dot(h[...], W1[...],
                            preferred_element_type=jnp.float32) + b1[...], 0.0)
    out_ref[...] = jnp.dot(t, W2[...],
                           preferred_element_type=jnp.float32) + b2[...]


def _tc_dec(h, d):
    args = (h, d["W1"], d["b1"], d["W2"], d["b2"])
    return pl.pallas_call(
        _dec_body,
        grid=(N4 // BN4,),
        in_specs=[_rows(BN4, 128)] + [_full(a) for a in args[1:]],
        out_specs=pl.BlockSpec((BN4, 8), lambda i: (i, 0)),
        out_shape=jax.ShapeDtypeStruct((N4, 8), jnp.float32),
    )(*args)


def _bd4(W):
    return jax.scipy.linalg.block_diag(W, W, W, W)


def _t4(v):
    return jnp.tile(v.reshape(1, -1), (1, 4))


def _prep_mlp4(pr):
    return {"W1": _bd4(pr["W1"]), "b1": _t4(pr["b1"]),
            "W2": _bd4(pr["W2"]), "b2": _t4(pr["b2"]),
            "g": _t4(pr["g"]), "beta": _t4(pr["beta"])}


def kernel(x, edge_index, edge_attr, p, mean_vec_x, std_vec_x,
           mean_vec_edge, std_vec_edge, params):
    pad = E_PAD - E
    src = edge_index[0]
    dst = edge_index[1]
    src_g1 = jnp.concatenate([src, jnp.zeros((pad,), jnp.int32)])
    dst_g1 = jnp.concatenate([dst, jnp.zeros((pad,), jnp.int32)])
    src_s1 = jnp.concatenate([src, jnp.full((pad,), N, jnp.int32)])
    src_gh = (src_g1[:EH], src_g1[EH:])
    dst_gh = (dst_g1[:EH], dst_g1[EH:])
    src_sh = (src_s1[:EH], src_s1[EH:])
    ea4 = jnp.concatenate(
        [edge_attr, jnp.zeros((pad, edge_attr.shape[1]), jnp.float32)]
    ).reshape(E4, 4 * edge_attr.shape[1])
    x4 = x.reshape(N4, 4 * x.shape[1])
    bdo = _bd4(jnp.full((HID, HID), 1.0 / HID, jnp.float32))

    h4 = _tc_encoder(x4, _t4(mean_vec_x), _t4(std_vec_x),
                     _prep_mlp4(params["node_enc"]), bdo, BN4)
    enc_e = _prep_mlp4(params["edge_enc"])
    e4h = tuple(
        _tc_encoder(ea4[h * E4H:(h + 1) * E4H], _t4(mean_vec_edge),
                    _t4(std_vec_edge), enc_e, bdo, BE4)
        for h in range(2))

    for lp in params["layers"]:
        em = lp["edge_mlp"]
        ew = {"W1a": _bd4(em["W1"][:HID]), "W1b": _bd4(em["W1"][HID:2 * HID]),
              "W1c": _bd4(em["W1"][2 * HID:]), "b1": _t4(em["b1"]),
              "W2": _bd4(em["W2"]), "b2": _t4(em["b2"]),
              "g": _t4(em["g"]), "beta": _t4(em["beta"])}
        nm = lp["node_mlp"]
        nw = {"W1a": _bd4(nm["W1"][:HID]), "W1b": _bd4(nm["W1"][HID:]),
              "b1": _t4(nm["b1"]), "W2": _bd4(nm["W2"]), "b2": _t4(nm["b2"]),
              "g": _t4(nm["g"]), "beta": _t4(nm["beta"])}
        h_lin = h4.reshape(N, HID)
        upd4h = []
        aggs = []
        for hf in range(2):
            ga, gb = _sc_gather(h_lin, dst_gh[hf], src_gh[hf])
            upd4h.append(_tc_edge(ga.reshape(E4H, 128), gb.reshape(E4H, 128),
                                  e4h[hf], ew, bdo))
            aggs.append(_sc_scatter(upd4h[hf].reshape(EH, HID), src_sh[hf]))
        h4 = _tc_node(h4, aggs[0].reshape(N4, 128), aggs[1].reshape(N4, 128),
                      nw, bdo)
        e4h = tuple(upd4h)

    d = params["dec"]
    out4 = _tc_dec(h4, {"W1": _bd4(d["W1"]), "b1": _t4(d["b1"]),
                        "W2": _bd4(d["W2"]), "b2": _t4(d["b2"])})
    return out4.reshape(N, 2)
